# baseline (speedup 1.0000x reference)
import functools

import jax
import jax.numpy as jnp
from jax.experimental import pallas as pl
from jax.experimental.pallas import tpu as pltpu


def _pick_block(n, target=1000):
    best = None
    for b in range(8, target + 1, 8):
        if n % b == 0:
            best = b
    return best


def _v2e_kernel(x_ref, hg_ref, w1_ref, b1_ref, w2_ref, b2_ref, out_ref,
                de_ref):
    n = pl.program_id(0)
    nsteps = pl.num_programs(0)
    h = hg_ref[...]
    x1 = jnp.dot(x_ref[...], w1_ref[...],
                 preferred_element_type=jnp.float32) + b1_ref[...]
    part = jax.lax.dot_general(
        h.astype(jnp.bfloat16), x1.astype(jnp.bfloat16),
        (((0,), (0,)), ((), ())),
        preferred_element_type=jnp.float32)
    de_part = jnp.sum(h, axis=0, keepdims=True)

    @pl.when(n == 0)
    def _init():
        out_ref[...] = part
        de_ref[...] = de_part

    @pl.when(n > 0)
    def _acc():
        out_ref[...] += part
        de_ref[...] += de_part

    @pl.when(n == nsteps - 1)
    def _finish():
        scale = jnp.transpose(1.0 / de_ref[...])
        y = jnp.maximum(out_ref[...] * scale, 0.0)
        out_ref[...] = jnp.dot(
            y.astype(jnp.bfloat16), w2_ref[...].astype(jnp.bfloat16),
            preferred_element_type=jnp.float32) + b2_ref[...]


def _e2v_kernel(hg_ref, y_ref, out_ref):
    h = hg_ref[...]
    xv = jax.lax.dot_general(
        h.astype(jnp.bfloat16), y_ref[...],
        (((1,), (0,)), ((), ())),
        preferred_element_type=jnp.float32)
    dv = jnp.sum(h, axis=1, keepdims=True)
    scale = jnp.where(dv > 0.0, 1.0 / dv, 0.0)
    out_ref[...] = jnp.maximum(xv * scale, 0.0)


@jax.jit
def kernel(X, hg, W_v2e, b_v2e, W_e2v, b_e2v):
    N, C = X.shape
    M = hg.shape[1]
    blk1 = _pick_block(N)
    blk2 = _pick_block(N)

    b1 = b_v2e.reshape(1, C)
    b2 = b_e2v.reshape(1, C)

    y = pl.pallas_call(
        _v2e_kernel,
        grid=(N // blk1,),
        in_specs=[
            pl.BlockSpec((blk1, C), lambda n: (n, 0)),
            pl.BlockSpec((blk1, M), lambda n: (n, 0)),
            pl.BlockSpec((C, C), lambda n: (0, 0)),
            pl.BlockSpec((1, C), lambda n: (0, 0)),
            pl.BlockSpec((C, C), lambda n: (0, 0)),
            pl.BlockSpec((1, C), lambda n: (0, 0)),
        ],
        out_specs=pl.BlockSpec((M, C), lambda n: (0, 0)),
        out_shape=jax.ShapeDtypeStruct((M, C), jnp.float32),
        scratch_shapes=[pltpu.VMEM((1, M), jnp.float32)],
        compiler_params=pltpu.CompilerParams(
            dimension_semantics=("arbitrary",)),
    )(X, hg, W_v2e, b1, W_e2v, b2)

    y16 = y.astype(jnp.bfloat16)

    xv = pl.pallas_call(
        _e2v_kernel,
        grid=(N // blk2,),
        in_specs=[
            pl.BlockSpec((blk2, M), lambda n: (n, 0)),
            pl.BlockSpec((M, C), lambda n: (0, 0)),
        ],
        out_specs=pl.BlockSpec((blk2, C), lambda n: (n, 0)),
        out_shape=jax.ShapeDtypeStruct((N, C), jnp.float32),
        compiler_params=pltpu.CompilerParams(
            dimension_semantics=("arbitrary",)),
    )(hg, y16)

    return xv


# speedup vs baseline: 1.0263x; 1.0263x over previous
import jax
import jax.numpy as jnp
from jax.experimental import pallas as pl
from jax.experimental.pallas import tpu as pltpu


def _pick_block(n, target=1000):
    best = None
    for b in range(8, target + 1, 8):
        if n % b == 0:
            best = b
    return best


def _fused_kernel(x_ref, hg_ref, w1_ref, b1_ref, w2_ref, b2_ref, out_ref,
                  acc_ref, y_ref):
    s = pl.program_id(0)
    n = pl.program_id(1)
    nsteps = pl.num_programs(1)
    C = w1_ref.shape[0]
    blk = x_ref.shape[0]

    @pl.when(s == 0)
    def _v2e():
        h16 = hg_ref[...].astype(jnp.bfloat16)
        x1 = jnp.dot(x_ref[...], w1_ref[...],
                     preferred_element_type=jnp.float32) + b1_ref[...]
        x1aug = jnp.concatenate(
            [x1, jnp.ones((blk, 8), jnp.float32)], axis=1).astype(jnp.bfloat16)
        part = jax.lax.dot_general(
            x1aug, h16, (((0,), (0,)), ((), ())),
            preferred_element_type=jnp.float32)

        @pl.when(n == 0)
        def _init():
            acc_ref[...] = part

        @pl.when(n > 0)
        def _acc():
            acc_ref[...] += part

        @pl.when(n == nsteps - 1)
        def _finish():
            de = acc_ref[C:C + 1, :]
            y = jnp.maximum(acc_ref[:C, :] * (1.0 / de), 0.0)
            y2 = jax.lax.dot_general(
                w2_ref[...].astype(jnp.bfloat16), y.astype(jnp.bfloat16),
                (((0,), (0,)), ((), ())),
                preferred_element_type=jnp.float32) + b2_ref[...]
            y_ref[...] = jnp.transpose(y2).astype(jnp.bfloat16)

    @pl.when(s == 1)
    def _e2v():
        h = hg_ref[...]
        xv = jnp.dot(h.astype(jnp.bfloat16), y_ref[...],
                     preferred_element_type=jnp.float32)
        dv = jnp.sum(h, axis=1, keepdims=True)
        scale = jnp.where(dv > 0.0, 1.0 / dv, 0.0)
        out_ref[...] = jnp.maximum(xv * scale, 0.0)


@jax.jit
def kernel(X, hg, W_v2e, b_v2e, W_e2v, b_e2v):
    N, C = X.shape
    M = hg.shape[1]
    blk = _pick_block(N)

    b1 = b_v2e.reshape(1, C)
    b2 = b_e2v.reshape(C, 1)

    xv = pl.pallas_call(
        _fused_kernel,
        grid=(2, N // blk),
        in_specs=[
            pl.BlockSpec((blk, C), lambda s, n: (n, 0)),
            pl.BlockSpec((blk, M), lambda s, n: (n, 0)),
            pl.BlockSpec((C, C), lambda s, n: (0, 0)),
            pl.BlockSpec((1, C), lambda s, n: (0, 0)),
            pl.BlockSpec((C, C), lambda s, n: (0, 0)),
            pl.BlockSpec((C, 1), lambda s, n: (0, 0)),
        ],
        out_specs=pl.BlockSpec((blk, C), lambda s, n: (n, 0)),
        out_shape=jax.ShapeDtypeStruct((N, C), jnp.float32),
        scratch_shapes=[
            pltpu.VMEM((C + 8, M), jnp.float32),
            pltpu.VMEM((M, C), jnp.bfloat16),
        ],
        compiler_params=pltpu.CompilerParams(
            dimension_semantics=("arbitrary", "arbitrary")),
    )(X, hg, W_v2e, b1, W_e2v, b2)

    return xv
